# Initial kernel scaffold; baseline (speedup 1.0000x reference)
import jax
import jax.numpy as jnp
from jax.experimental import pallas as pl


def kernel(x, table):
    raise NotImplementedError("write your pallas kernel here")


# speedup vs baseline: 1.0943x; 1.0943x over previous
import functools

import jax
import jax.numpy as jnp
from jax import lax
from jax.experimental import pallas as pl
from jax.experimental.pallas import tpu as pltpu
from jax.experimental.pallas import tpu_sc as plsc

_INFO = plsc.get_sparse_core_info()
_NC, _NS = _INFO.num_cores, _INFO.num_subcores
_NW = _NC * _NS

_CHUNK = 1024


def _make_gather(n, v, d):
    assert n % _NW == 0
    per_w = n // _NW
    assert per_w % _CHUNK == 0
    n_chunks = per_w // _CHUNK
    mesh = plsc.VectorSubcoreMesh(core_axis_name="c", subcore_axis_name="s")

    @functools.partial(
        pl.kernel,
        mesh=mesh,
        out_type=jax.ShapeDtypeStruct((n, d), jnp.float32),
        compiler_params=pltpu.CompilerParams(use_tc_tiling_on_sc=False),
        scratch_types=[
            pltpu.VMEM((_CHUNK,), jnp.int32),
            pltpu.VMEM((_CHUNK, d), jnp.float32),
            pltpu.SemaphoreType.DMA,
        ],
    )
    def gather(idx_hbm, table_hbm, out_hbm, idx_v, rows_v, sem):
        wid = lax.axis_index("s") * _NC + lax.axis_index("c")
        base = wid * per_w

        def body(i, carry):
            off = base + i * _CHUNK
            pltpu.sync_copy(idx_hbm.at[pl.ds(off, _CHUNK)], idx_v)
            pltpu.async_copy(table_hbm.at[idx_v], rows_v, sem).wait()
            pltpu.sync_copy(rows_v, out_hbm.at[pl.ds(off, _CHUNK)])
            return carry

        lax.fori_loop(0, n_chunks, body, 0)

    return gather


def kernel(x, table):
    b, l = x.shape
    v, d = table.shape
    flat = x.reshape(b * l).astype(jnp.int32)
    out = _make_gather(b * l, v, d)(flat, table)
    return out.reshape(b, l, d)


# speedup vs baseline: 1.1142x; 1.0181x over previous
import functools

import jax
import jax.numpy as jnp
from jax import lax
from jax.experimental import pallas as pl
from jax.experimental.pallas import tpu as pltpu
from jax.experimental.pallas import tpu_sc as plsc

_INFO = plsc.get_sparse_core_info()
_NC, _NS = _INFO.num_cores, _INFO.num_subcores
_NW = _NC * _NS

_CHUNK = 1280


def _make_gather(n, v, d):
    assert n % _NW == 0
    per_w = n // _NW
    assert per_w % _CHUNK == 0
    n_chunks = per_w // _CHUNK
    mesh = plsc.VectorSubcoreMesh(core_axis_name="c", subcore_axis_name="s")

    @functools.partial(
        pl.kernel,
        mesh=mesh,
        out_type=jax.ShapeDtypeStruct((n, d), jnp.float32),
        compiler_params=pltpu.CompilerParams(use_tc_tiling_on_sc=False),
        scratch_types=[
            pltpu.VMEM((n_chunks, _CHUNK), jnp.int32),
            pltpu.VMEM((2, _CHUNK, d), jnp.float32),
            pltpu.SemaphoreType.DMA,
            pltpu.SemaphoreType.DMA,
        ],
    )
    def gather(idx_hbm, table_hbm, out_hbm, idx_v, rows_v, gsem, wsem):
        wid = lax.axis_index("s") * _NC + lax.axis_index("c")
        base = wid * per_w

        pltpu.sync_copy(idx_hbm.at[wid], idx_v)

        row_bytes = _CHUNK * d * 4

        def start_gather(c, buf):
            pltpu.async_copy(table_hbm.at[idx_v.at[c]], rows_v.at[buf], gsem)

        def wait(sem, buf):
            pltpu.make_async_copy(
                table_hbm.at[pl.ds(0, _CHUNK)], rows_v.at[buf], sem
            ).wait()

        def start_write(c, buf):
            pltpu.async_copy(
                rows_v.at[buf], out_hbm.at[pl.ds(base + c * _CHUNK, _CHUNK)], wsem
            )

        assert n_chunks % 2 == 0
        n_pairs = n_chunks // 2

        start_gather(0, 0)

        def body(g, carry):
            c0 = 2 * g

            @pl.when(g >= 1)
            def _():
                wait(wsem, 1)

            start_gather(c0 + 1, 1)
            wait(gsem, 0)
            start_write(c0, 0)

            wait(wsem, 0)

            @pl.when(g + 1 < n_pairs)
            def _():
                start_gather(c0 + 2, 0)

            wait(gsem, 1)
            start_write(c0 + 1, 1)
            return carry

        lax.fori_loop(0, n_pairs, body, 0)
        wait(wsem, 1)

    return gather


def kernel(x, table):
    b, l = x.shape
    v, d = table.shape
    n = b * l
    flat = x.reshape(n).astype(jnp.int32)
    per_w = n // _NW
    idx3 = flat.reshape(_NW, per_w // _CHUNK, _CHUNK)
    out = _make_gather(n, v, d)(idx3, table)
    return out.reshape(b, l, d)
